# baseline (speedup 1.0000x reference)
import jax
import jax.numpy as jnp
from jax import lax
from jax.experimental import pallas as pl
from jax.experimental.pallas import tpu as pltpu
from jax.experimental.pallas import tpu_sc as plsc

B = 8
H = W = 112
C = 96
IS = H * W * C
OUT4 = 4 * IS
NREG = 3
R = OUT4 // NREG
NTILES = 16
SLABS_PER_TILE = H // NTILES
QROWS = 24
CHUNK_VECS = QROWS * (W // 16)
NCHUNK = SLABS_PER_TILE * 4
RING_ROWS = 64
LANE_SLOTS = RING_ROWS * 8
DRAIN_LAG = 8
WORDS_PER_TILE = R // NTILES
ZCHUNK = 1024
NZ = WORDS_PER_TILE // ZCHUNK
TASKS_PER_CORE = 4 * NREG


def _unpool_body(x0_hbm, idx_hbm, out_hbm,
                 acc_sh, idx_c0, val_c0, idx_c1, val_c1, cb_idx, cb_val, zbuf,
                 sem_in0, sem_in1, sem_add, sem_out):
    core = lax.axis_index("c")
    sub = lax.axis_index("s")
    lane = lax.iota(jnp.int32, 16)
    zeros16 = jnp.zeros((16,), jnp.float32)
    r_u32 = jnp.full((16,), R, jnp.uint32)

    def _zb(i, _):
        zbuf[pl.ds(i * 16, 16)] = zeros16
        return 0
    lax.fori_loop(0, ZCHUNK // 16, _zb, 0)

    def _zero_acc():
        def _z(k, _):
            pltpu.async_copy(zbuf, acc_sh.at[pl.ds(sub * WORDS_PER_TILE + k * ZCHUNK, ZCHUNK)], sem_out)
            return 0
        lax.fori_loop(0, NZ, _z, 0)
        def _zw(k, _):
            pltpu.make_async_copy(zbuf, acc_sh.at[pl.ds(sub * WORDS_PER_TILE + k * ZCHUNK, ZCHUNK)], sem_out).wait()
            return 0
        lax.fori_loop(0, NZ, _zw, 0)

    _zero_acc()
    plsc.subcore_barrier()

    def _fire_one(j, _):
        jr = j & (RING_ROWS - 1)
        pltpu.async_copy(cb_val.at[jr], acc_sh.at[cb_idx.at[jr]], sem_add, add=True)
        return 0

    def _drain_one(j, _):
        jr = j & (RING_ROWS - 1)
        pltpu.make_async_copy(cb_val.at[jr], acc_sh.at[cb_idx.at[jr]], sem_add).wait()
        return 0

    def _task(t, _):
        b = core * 4 + t // NREG
        r = t % NREG
        lo = r * R

        def _compact(idx_chunk, val_chunk, state):
            cnt, fired, drained = state
            lim = jnp.full((16,), 0, jnp.int32) + (drained * 8 + LANE_SLOTS)

            def _crow(cr, cnt):
                for wg in range(W // 16):
                    idx = idx_chunk[cr, pl.ds(wg * 16, 16)]
                    val = val_chunk[cr, pl.ds(wg * 16, 16)]
                    idxl = idx - lo
                    m = plsc.bitcast(idxl, jnp.uint32) < r_u32
                    m = m & (cnt < lim)
                    c9 = cnt & (LANE_SLOTS - 1)
                    row = lax.shift_right_logical(c9, 3)
                    col = ((c9 & 7) << 4) | lane
                    plsc.store_scatter(cb_idx, [row, col], idxl, mask=m)
                    plsc.store_scatter(cb_val, [row, col], val, mask=m)
                    cnt = cnt + m.astype(jnp.int32)
                return cnt

            cnt = lax.fori_loop(0, QROWS, _crow, cnt)
            target = jnp.min(cnt) >> 3
            lax.fori_loop(fired, target, _fire_one, 0)
            need = jnp.maximum(drained, target - DRAIN_LAG)
            lax.fori_loop(drained, need, _drain_one, 0)
            return cnt, target, need

        def _load(k, buf_i, buf_v, sem):
            h = sub * SLABS_PER_TILE + lax.shift_right_logical(k, 2)
            c0 = (k & 3) * QROWS
            pltpu.async_copy(idx_hbm.at[b, h, pl.ds(c0, QROWS)], buf_i, sem)
            pltpu.async_copy(x0_hbm.at[b, h, pl.ds(c0, QROWS)], buf_v, sem)

        def _wait(k, buf_i, buf_v, sem):
            h = sub * SLABS_PER_TILE + lax.shift_right_logical(k, 2)
            c0 = (k & 3) * QROWS
            pltpu.make_async_copy(idx_hbm.at[b, h, pl.ds(c0, QROWS)], buf_i, sem).wait()
            pltpu.make_async_copy(x0_hbm.at[b, h, pl.ds(c0, QROWS)], buf_v, sem).wait()

        _load(0, idx_c0, val_c0, sem_in0)

        def _pair(p, state):
            k0 = p * 2
            _load(k0 + 1, idx_c1, val_c1, sem_in1)
            _wait(k0, idx_c0, val_c0, sem_in0)
            state = _compact(idx_c0, val_c0, state)
            nxt = jnp.minimum(k0 + 2, NCHUNK - 1)
            _load(nxt, idx_c0, val_c0, sem_in0)
            _wait(k0 + 1, idx_c1, val_c1, sem_in1)
            state = _compact(idx_c1, val_c1, state)
            return state

        state = lax.fori_loop(0, NCHUNK // 2, _pair,
                              (jnp.zeros((16,), jnp.int32), jnp.int32(0), jnp.int32(0)))
        cnt, fired, drained = state
        _wait(NCHUNK - 1, idx_c0, val_c0, sem_in0)

        maxrows = (jnp.max(cnt) + 7) >> 3
        def _holes(j, _):
            jr = j & (RING_ROWS - 1)
            for cg in range(8):
                s = j * 8 + cg
                hm = (jnp.full((16,), 0, jnp.int32) + s) >= cnt
                cols = cg * 16 + lane
                rsp = jnp.full((16,), 0, jnp.int32) + jr
                plsc.store_scatter(cb_idx, [rsp, cols], (s * 16 + lane) * 8, mask=hm)
                plsc.store_scatter(cb_val, [rsp, cols], zeros16, mask=hm)
            return 0
        lax.fori_loop(fired, maxrows, _holes, 0)
        lax.fori_loop(fired, maxrows, _fire_one, 0)
        lax.fori_loop(drained, maxrows, _drain_one, 0)

        plsc.subcore_barrier()

        out_base = b * OUT4 + r * R + sub * WORDS_PER_TILE
        pltpu.sync_copy(acc_sh.at[pl.ds(sub * WORDS_PER_TILE, WORDS_PER_TILE)],
                        out_hbm.at[pl.ds(out_base, WORDS_PER_TILE)])
        _zero_acc()
        plsc.subcore_barrier()
        return 0

    lax.fori_loop(0, TASKS_PER_CORE, _task, 0)


@jax.jit
def _unpool(x0_t, idx_t):
    mesh = plsc.VectorSubcoreMesh(core_axis_name="c", subcore_axis_name="s")
    f = pl.kernel(
        _unpool_body,
        out_type=jax.ShapeDtypeStruct((B * OUT4,), jnp.float32),
        mesh=mesh,
        compiler_params=pltpu.CompilerParams(needs_layout_passes=False),
        scratch_types=[
            pltpu.VMEM_SHARED((R,), jnp.float32),
            pltpu.VMEM((QROWS, W), jnp.int32),
            pltpu.VMEM((QROWS, W), jnp.float32),
            pltpu.VMEM((QROWS, W), jnp.int32),
            pltpu.VMEM((QROWS, W), jnp.float32),
            pltpu.VMEM((RING_ROWS, 128), jnp.int32),
            pltpu.VMEM((RING_ROWS, 128), jnp.float32),
            pltpu.VMEM((ZCHUNK,), jnp.float32),
            pltpu.SemaphoreType.DMA,
            pltpu.SemaphoreType.DMA,
            pltpu.SemaphoreType.DMA,
            pltpu.SemaphoreType.DMA,
        ],
    )
    return f(x0_t, idx_t)


def kernel(x_0, x_1):
    x0_t = jnp.transpose(x_0, (0, 1, 3, 2))
    idx_t = jnp.transpose(x_1.astype(jnp.int32), (0, 1, 3, 2))
    out = _unpool(x0_t, idx_t)
    return out.reshape(B, 2 * H, 2 * W, C)
